# baseline (speedup 1.0000x reference)
import jax
import jax.numpy as jnp
import numpy as np
from jax.experimental import pallas as pl
from jax.experimental.pallas import tpu as pltpu

_BF = jnp.bfloat16
_F32 = jnp.float32
_B = 2


def _tower_kernel(xr_ref,
                  w0_ref, s0_ref, t0_ref,
                  w1_ref, s1_ref, t1_ref,
                  w2_ref, s2_ref, t2_ref,
                  w3_ref, s3_ref, t3_ref,
                  out_ref,
                  lhs0, lhs1, lhs2, lhs3, ys, pad1, pad2, pad3):
    pad1[...] = jnp.zeros_like(pad1)
    pad2[...] = jnp.zeros_like(pad2)
    pad3[...] = jnp.zeros_like(pad3)

    def conv_pool(b, lhs_s, w_ref, s_ref, t_ref, G, H, K):
        M = G * H
        y = jnp.dot(lhs_s[b].reshape(M, K), w_ref[...],
                    preferred_element_type=_F32)
        y = jnp.maximum(y * s_ref[...] + t_ref[...], 0.0)
        ys[b * 576:b * 576 + M, :] = jnp.maximum(y[:, :128], y[:, 128:])
        a = jnp.maximum(ys[pl.ds(b * 576, M // 2, 2), :],
                        ys[pl.ds(b * 576 + 1, M // 2, 2), :])
        return a.astype(_BF)

    for b in range(_B):
        xr = xr_ref[b]
        for dy in range(3):
            for g in range(6):
                lhs0[b, g, :, dy * 54:(dy + 1) * 54] = \
                    xr[dy:dy + 96, 48 * g:48 * g + 54]
        p = conv_pool(b, lhs0, w0_ref, s0_ref, t0_ref, 6, 96, 162)
        br = p.reshape(6, 48, 128)
        pad1[b, 1:49, 16:784] = jnp.concatenate(
            [br[i] for i in range(6)], axis=-1)

        for dy in range(3):
            for g in range(6):
                lhs1[b, g, :, dy * 160:(dy + 1) * 160] = \
                    pad1[b, dy:dy + 48, 128 * g:128 * g + 160]
        p = conv_pool(b, lhs1, w1_ref, s1_ref, t1_ref, 6, 48, 480)
        br = p.reshape(6, 24, 128)
        pad2[b, 1:25, 32:800] = jnp.concatenate(
            [br[i] for i in range(6)], axis=-1)

        for dy in range(3):
            for g in range(6):
                lhs2[b, g, :, dy * 192:(dy + 1) * 192] = \
                    pad2[b, dy:dy + 24, 128 * g:128 * g + 192]
        p = conv_pool(b, lhs2, w2_ref, s2_ref, t2_ref, 6, 24, 576)
        br = p.reshape(6, 12, 128)
        pad3[b, 1:13, 64:832] = jnp.concatenate(
            [br[i] for i in range(6)], axis=-1)

        for dy in range(3):
            for g in range(6):
                lhs3[b, g, :, dy * 256:(dy + 1) * 256] = \
                    pad3[b, dy:dy + 12, 128 * g:128 * g + 256]
        p = conv_pool(b, lhs3, w3_ref, s3_ref, t3_ref, 6, 12, 768)
        out_ref[b] = jnp.transpose(p.reshape(6, 6, 128),
                                   (1, 0, 2)).reshape(36, 128)


def _conv_tower(xr, wp, N):
    const2 = lambda n: (0, 0)
    in_specs = [pl.BlockSpec((_B, 98, 294), lambda n: (n, 0, 0))]
    for i in range(4):
        in_specs += [pl.BlockSpec(wp[f"w{i}"].shape, const2),
                     pl.BlockSpec((1, 256), const2),
                     pl.BlockSpec((1, 256), const2)]
    return pl.pallas_call(
        _tower_kernel,
        out_shape=jax.ShapeDtypeStruct((N, 36, 128), _BF),
        grid=(N // _B,),
        in_specs=in_specs,
        out_specs=pl.BlockSpec((_B, 36, 128), lambda n: (n, 0, 0)),
        scratch_shapes=[
            pltpu.VMEM((_B, 6, 96, 162), _BF),
            pltpu.VMEM((_B, 6, 48, 480), _BF),
            pltpu.VMEM((_B, 6, 24, 576), _BF),
            pltpu.VMEM((_B, 6, 12, 768), _BF),
            pltpu.VMEM((_B * 576, 128), _F32),
            pltpu.VMEM((_B, 50, 800), _BF),
            pltpu.VMEM((_B, 26, 832), _BF),
            pltpu.VMEM((_B, 14, 896), _BF),
        ],
        compiler_params=pltpu.CompilerParams(
            dimension_semantics=("parallel",),
            vmem_limit_bytes=32 * 1024 * 1024),
    )(xr, wp["w0"], wp["s0"], wp["t0"], wp["w1"], wp["s1"], wp["t1"],
      wp["w2"], wp["s2"], wp["t2"], wp["w3"], wp["s3"], wp["t3"])


def _head_kernel(x_ref, w0_ref, b0_ref, w1_ref, b1_ref, w2_ref, b2_ref,
                 o_ref):
    h = jnp.maximum(
        jnp.dot(x_ref[...], w0_ref[...], preferred_element_type=_F32)
        + b0_ref[...], 0.0)
    h = jnp.maximum(
        jnp.dot(h.astype(_BF), w1_ref[...], preferred_element_type=_F32)
        + b1_ref[...], 0.0)
    z = jnp.dot(h.astype(_BF), w2_ref[...], preferred_element_type=_F32) \
        + b2_ref[...]
    o_ref[...] = 1.0 / (1.0 + jnp.exp(-z))


def _head(feat, w0, b0, w1, b1, w2, b2):
    N = feat.shape[0]
    vmem = pl.BlockSpec(memory_space=pltpu.MemorySpace.VMEM)
    return pl.pallas_call(
        _head_kernel,
        out_shape=jax.ShapeDtypeStruct((N, 1), _F32),
        in_specs=[vmem] * 7,
        out_specs=vmem,
    )(feat, w0, b0, w1, b1, w2, b2)


def _pack_idx(P, cin):
    L = P + 2
    R = 3 * L * cin
    idx = np.zeros((R, P), np.int32)
    msk = np.zeros((R, P), np.bool_)
    order = list(range(0, P, 2)) + list(range(1, P, 2))
    for col, p in enumerate(order):
        for dy in range(3):
            for dx in range(3):
                for ci in range(cin):
                    r = dy * L * cin + (p + dx) * cin + ci
                    idx[r, col] = (dy * 3 + dx) * cin + ci
                    msk[r, col] = True
    return jnp.asarray(idx), jnp.asarray(msk)


def _pack_conv(w, P):
    cin, cout = w.shape[2], w.shape[3]
    idx, msk = _pack_idx(P, cin)
    g = w.reshape(9 * cin, cout)[idx]
    g = jnp.where(msk[:, :, None], g, 0)
    return g.reshape(idx.shape[0], P * cout).astype(_BF)


def _impl(x_nchw, w0c, s0, t0, w_c1, s1, t1, w_c2, s2, t2, w_c3, s3, t3,
          w0, b0, w1, b1, w2, b2):
    N = x_nchw.shape[0]
    x = jnp.transpose(x_nchw, (0, 2, 3, 1))
    xp = jnp.pad(x, ((0, 0), (1, 1), (1, 1), (0, 0)))
    xr = xp.reshape(N, 98, 294).astype(_BF)

    wp = {
        "w0": _pack_conv(w0c.reshape(3, 3, 3, 16), 16),
        "w1": _pack_conv(w_c1, 8),
        "w2": _pack_conv(w_c2, 4),
        "w3": _pack_conv(w_c3, 2),
        "s0": jnp.tile(s0, (1, 16)), "t0": jnp.tile(t0, (1, 16)),
        "s1": jnp.tile(s1, (1, 8)), "t1": jnp.tile(t1, (1, 8)),
        "s2": jnp.tile(s2, (1, 4)), "t2": jnp.tile(t2, (1, 4)),
        "s3": jnp.tile(s3, (1, 2)), "t3": jnp.tile(t3, (1, 2)),
    }
    feat = _conv_tower(xr, wp, N).reshape(N, 36 * 128)
    return _head(feat, w0, b0, w1, b1, w2, b2)


_forward = jax.jit(_impl)


def kernel(x_nchw, w0c, s0, t0, w_c1, s1, t1, w_c2, s2, t2, w_c3, s3, t3,
           w0, b0, w1, b1, w2, b2):
    return _forward(x_nchw, w0c, s0, t0, w_c1, s1, t1, w_c2, s2, t2,
                    w_c3, s3, t3, w0, b0, w1, b1, w2, b2)


# speedup vs baseline: 1.0343x; 1.0343x over previous
import jax
import jax.numpy as jnp
import numpy as np
from jax.experimental import pallas as pl
from jax.experimental.pallas import tpu as pltpu

_BF = jnp.bfloat16
_F32 = jnp.float32
_B = 4


def _tower_kernel(xr_ref,
                  w0_ref, t0_ref, w1_ref, t1_ref,
                  w2_ref, t2_ref, w3_ref, t3_ref,
                  out_ref,
                  lhs0, lhs1, lhs2, lhs3, ys, pad1, pad2, pad3):
    pad1[...] = jnp.zeros_like(pad1)
    pad2[...] = jnp.zeros_like(pad2)
    pad3[...] = jnp.zeros_like(pad3)

    def conv_pool(b, lhs_s, w_ref, t_ref, G, H, K):
        M = G * H
        y = jnp.dot(lhs_s[b].reshape(M, K), w_ref[...],
                    preferred_element_type=_F32)
        y = jnp.maximum(y + t_ref[...], 0.0)
        ys[b * 576:b * 576 + M, :] = jnp.maximum(y[:, :128], y[:, 128:])
        a = jnp.maximum(ys[pl.ds(b * 576, M // 2, 2), :],
                        ys[pl.ds(b * 576 + 1, M // 2, 2), :])
        return a.astype(_BF)

    for b in range(_B):
        xr = xr_ref[b]
        for dy in range(3):
            for g in range(6):
                lhs0[b, g, :, dy * 54:(dy + 1) * 54] = \
                    xr[dy:dy + 96, 48 * g:48 * g + 54]
        p = conv_pool(b, lhs0, w0_ref, t0_ref, 6, 96, 162)
        br = p.reshape(6, 48, 128)
        pad1[b, 1:49, 16:784] = jnp.concatenate(
            [br[i] for i in range(6)], axis=-1)

        for dy in range(3):
            for g in range(6):
                lhs1[b, g, :, dy * 160:(dy + 1) * 160] = \
                    pad1[b, dy:dy + 48, 128 * g:128 * g + 160]
        p = conv_pool(b, lhs1, w1_ref, t1_ref, 6, 48, 480)
        br = p.reshape(6, 24, 128)
        pad2[b, 1:25, 32:800] = jnp.concatenate(
            [br[i] for i in range(6)], axis=-1)

        for dy in range(3):
            for g in range(6):
                lhs2[b, g, :, dy * 192:(dy + 1) * 192] = \
                    pad2[b, dy:dy + 24, 128 * g:128 * g + 192]
        p = conv_pool(b, lhs2, w2_ref, t2_ref, 6, 24, 576)
        br = p.reshape(6, 12, 128)
        pad3[b, 1:13, 64:832] = jnp.concatenate(
            [br[i] for i in range(6)], axis=-1)

        for dy in range(3):
            for g in range(6):
                lhs3[b, g, :, dy * 256:(dy + 1) * 256] = \
                    pad3[b, dy:dy + 12, 128 * g:128 * g + 256]
        p = conv_pool(b, lhs3, w3_ref, t3_ref, 6, 12, 768)
        out_ref[b] = jnp.transpose(p.reshape(6, 6, 128),
                                   (1, 0, 2)).reshape(36, 128)


def _conv_tower(xr, wp, N):
    const2 = lambda n: (0, 0)
    in_specs = [pl.BlockSpec((_B, 98, 294), lambda n: (n, 0, 0))]
    for i in range(4):
        in_specs += [pl.BlockSpec(wp[f"w{i}"].shape, const2),
                     pl.BlockSpec((1, 256), const2)]
    return pl.pallas_call(
        _tower_kernel,
        out_shape=jax.ShapeDtypeStruct((N, 36, 128), _BF),
        grid=(N // _B,),
        in_specs=in_specs,
        out_specs=pl.BlockSpec((_B, 36, 128), lambda n: (n, 0, 0)),
        scratch_shapes=[
            pltpu.VMEM((_B, 6, 96, 162), _BF),
            pltpu.VMEM((_B, 6, 48, 480), _BF),
            pltpu.VMEM((_B, 6, 24, 576), _BF),
            pltpu.VMEM((_B, 6, 12, 768), _BF),
            pltpu.VMEM((_B * 576, 128), _F32),
            pltpu.VMEM((_B, 50, 800), _BF),
            pltpu.VMEM((_B, 26, 832), _BF),
            pltpu.VMEM((_B, 14, 896), _BF),
        ],
        compiler_params=pltpu.CompilerParams(
            dimension_semantics=("parallel",),
            vmem_limit_bytes=32 * 1024 * 1024),
    )(xr, wp["w0"], wp["t0"], wp["w1"], wp["t1"],
      wp["w2"], wp["t2"], wp["w3"], wp["t3"])


def _head_kernel(x_ref, w0_ref, b0_ref, w1_ref, b1_ref, w2_ref, b2_ref,
                 o_ref):
    h = jnp.maximum(
        jnp.dot(x_ref[...], w0_ref[...], preferred_element_type=_F32)
        + b0_ref[...], 0.0)
    h = jnp.maximum(
        jnp.dot(h.astype(_BF), w1_ref[...], preferred_element_type=_F32)
        + b1_ref[...], 0.0)
    z = jnp.dot(h.astype(_BF), w2_ref[...], preferred_element_type=_F32) \
        + b2_ref[...]
    o_ref[...] = 1.0 / (1.0 + jnp.exp(-z))


def _head(feat, w0, b0, w1, b1, w2, b2):
    N = feat.shape[0]
    vmem = pl.BlockSpec(memory_space=pltpu.MemorySpace.VMEM)
    return pl.pallas_call(
        _head_kernel,
        out_shape=jax.ShapeDtypeStruct((N, 1), _F32),
        in_specs=[vmem] * 7,
        out_specs=vmem,
    )(feat, w0, b0, w1, b1, w2, b2)


def _pack_idx(P, cin):
    L = P + 2
    R = 3 * L * cin
    idx = np.zeros((R, P), np.int32)
    msk = np.zeros((R, P), np.bool_)
    order = list(range(0, P, 2)) + list(range(1, P, 2))
    for col, p in enumerate(order):
        for dy in range(3):
            for dx in range(3):
                for ci in range(cin):
                    r = dy * L * cin + (p + dx) * cin + ci
                    idx[r, col] = (dy * 3 + dx) * cin + ci
                    msk[r, col] = True
    return jnp.asarray(idx), jnp.asarray(msk)


def _pack_conv(w, P, s):
    cin, cout = w.shape[2], w.shape[3]
    ws = (w.astype(_F32) * s.reshape(1, 1, 1, cout)).astype(_BF)
    idx, msk = _pack_idx(P, cin)
    g = ws.reshape(9 * cin, cout)[idx]
    g = jnp.where(msk[:, :, None], g, 0)
    return g.reshape(idx.shape[0], P * cout)


def _impl(x_nchw, w0c, s0, t0, w_c1, s1, t1, w_c2, s2, t2, w_c3, s3, t3,
          w0, b0, w1, b1, w2, b2):
    N = x_nchw.shape[0]
    x = jnp.transpose(x_nchw.astype(_BF), (0, 2, 3, 1))
    xp = jnp.pad(x, ((0, 0), (1, 1), (1, 1), (0, 0)))
    xr = xp.reshape(N, 98, 294)

    wp = {
        "w0": _pack_conv(w0c.reshape(3, 3, 3, 16), 16, s0),
        "w1": _pack_conv(w_c1, 8, s1),
        "w2": _pack_conv(w_c2, 4, s2),
        "w3": _pack_conv(w_c3, 2, s3),
        "t0": jnp.tile(t0, (1, 16)), "t1": jnp.tile(t1, (1, 8)),
        "t2": jnp.tile(t2, (1, 4)), "t3": jnp.tile(t3, (1, 2)),
    }
    feat = _conv_tower(xr, wp, N).reshape(N, 36 * 128)
    return _head(feat, w0, b0, w1, b1, w2, b2)


_forward = jax.jit(_impl)


def kernel(x_nchw, w0c, s0, t0, w_c1, s1, t1, w_c2, s2, t2, w_c3, s3, t3,
           w0, b0, w1, b1, w2, b2):
    return _forward(x_nchw, w0c, s0, t0, w_c1, s1, t1, w_c2, s2, t2,
                    w_c3, s3, t3, w0, b0, w1, b1, w2, b2)


# speedup vs baseline: 1.3911x; 1.3449x over previous
import jax
import jax.numpy as jnp
import numpy as np
from jax.experimental import pallas as pl
from jax.experimental.pallas import tpu as pltpu

_BF = jnp.bfloat16
_F32 = jnp.float32
_B = 4


def _tower_kernel(xr_ref,
                  w0_ref, t0_ref, w1_ref, t1_ref,
                  w2_ref, t2_ref, w3_ref, t3_ref,
                  out_ref, *scr):
    for c in range(2):
        lhs0, lhs1, lhs2, lhs3, ys, pad1, pad2, pad3 = scr[8 * c:8 * c + 8]
        pad1[...] = jnp.zeros_like(pad1)
        pad2[...] = jnp.zeros_like(pad2)
        pad3[...] = jnp.zeros_like(pad3)

        def conv_pool(lhs_s, w_ref, t_ref, H, K):
            M = 12 * H
            y = jnp.dot(lhs_s[...].reshape(M, K), w_ref[...],
                        preferred_element_type=_F32)
            y = jnp.maximum(y + t_ref[...], 0.0)
            ys[0:M, :] = jnp.maximum(y[:, :128], y[:, 128:])
            a = jnp.maximum(ys[pl.ds(0, M // 2, 2), :],
                            ys[pl.ds(1, M // 2, 2), :])
            return a.astype(_BF).reshape(2, 6, H // 2, 128)

        for dy in range(3):
            for g in range(6):
                lhs0[:, g, :, dy * 54:(dy + 1) * 54] = \
                    xr_ref[2 * c:2 * c + 2, dy:dy + 96, 48 * g:48 * g + 54]
        br = conv_pool(lhs0, w0_ref, t0_ref, 96, 162)
        pad1[:, 1:49, 16:784] = jnp.concatenate(
            [br[:, i] for i in range(6)], axis=-1)

        for dy in range(3):
            for g in range(6):
                lhs1[:, g, :, dy * 160:(dy + 1) * 160] = \
                    pad1[:, dy:dy + 48, 128 * g:128 * g + 160]
        br = conv_pool(lhs1, w1_ref, t1_ref, 48, 480)
        pad2[:, 1:25, 32:800] = jnp.concatenate(
            [br[:, i] for i in range(6)], axis=-1)

        for dy in range(3):
            for g in range(6):
                lhs2[:, g, :, dy * 192:(dy + 1) * 192] = \
                    pad2[:, dy:dy + 24, 128 * g:128 * g + 192]
        br = conv_pool(lhs2, w2_ref, t2_ref, 24, 576)
        pad3[:, 1:13, 64:832] = jnp.concatenate(
            [br[:, i] for i in range(6)], axis=-1)

        for dy in range(3):
            for g in range(6):
                lhs3[:, g, :, dy * 256:(dy + 1) * 256] = \
                    pad3[:, dy:dy + 12, 128 * g:128 * g + 256]
        br = conv_pool(lhs3, w3_ref, t3_ref, 12, 768)
        out_ref[2 * c:2 * c + 2] = jnp.transpose(
            br, (0, 2, 1, 3)).reshape(2, 36, 128)


def _pair_scratches():
    return [
        pltpu.VMEM((2, 6, 96, 162), _BF),
        pltpu.VMEM((2, 6, 48, 480), _BF),
        pltpu.VMEM((2, 6, 24, 576), _BF),
        pltpu.VMEM((2, 6, 12, 768), _BF),
        pltpu.VMEM((1152, 128), _F32),
        pltpu.VMEM((2, 50, 800), _BF),
        pltpu.VMEM((2, 26, 832), _BF),
        pltpu.VMEM((2, 14, 896), _BF),
    ]


def _conv_tower(xr, wp, N):
    const2 = lambda n: (0, 0)
    in_specs = [pl.BlockSpec((_B, 98, 294), lambda n: (n, 0, 0))]
    for i in range(4):
        in_specs += [pl.BlockSpec(wp[f"w{i}"].shape, const2),
                     pl.BlockSpec((1, 256), const2)]
    return pl.pallas_call(
        _tower_kernel,
        out_shape=jax.ShapeDtypeStruct((N, 36, 128), _BF),
        grid=(N // _B,),
        in_specs=in_specs,
        out_specs=pl.BlockSpec((_B, 36, 128), lambda n: (n, 0, 0)),
        scratch_shapes=_pair_scratches() + _pair_scratches(),
        compiler_params=pltpu.CompilerParams(
            dimension_semantics=("parallel",),
            vmem_limit_bytes=32 * 1024 * 1024),
    )(xr, wp["w0"], wp["t0"], wp["w1"], wp["t1"],
      wp["w2"], wp["t2"], wp["w3"], wp["t3"])


def _head_kernel(x_ref, w0_ref, b0_ref, w1_ref, b1_ref, w2_ref, b2_ref,
                 o_ref):
    h = jnp.maximum(
        jnp.dot(x_ref[...], w0_ref[...], preferred_element_type=_F32)
        + b0_ref[...], 0.0)
    h = jnp.maximum(
        jnp.dot(h.astype(_BF), w1_ref[...], preferred_element_type=_F32)
        + b1_ref[...], 0.0)
    z = jnp.dot(h.astype(_BF), w2_ref[...], preferred_element_type=_F32) \
        + b2_ref[...]
    o_ref[...] = 1.0 / (1.0 + jnp.exp(-z))


def _head(feat, w0, b0, w1, b1, w2, b2):
    N = feat.shape[0]
    vmem = pl.BlockSpec(memory_space=pltpu.MemorySpace.VMEM)
    return pl.pallas_call(
        _head_kernel,
        out_shape=jax.ShapeDtypeStruct((N, 1), _F32),
        in_specs=[vmem] * 7,
        out_specs=vmem,
    )(feat, w0, b0, w1, b1, w2, b2)


def _pack_idx(P, cin):
    L = P + 2
    R = 3 * L * cin
    idx = np.zeros((R, P), np.int32)
    msk = np.zeros((R, P), np.bool_)
    order = list(range(0, P, 2)) + list(range(1, P, 2))
    for col, p in enumerate(order):
        for dy in range(3):
            for dx in range(3):
                for ci in range(cin):
                    r = dy * L * cin + (p + dx) * cin + ci
                    idx[r, col] = (dy * 3 + dx) * cin + ci
                    msk[r, col] = True
    return jnp.asarray(idx), jnp.asarray(msk)


def _pack_conv(w, P, s):
    cin, cout = w.shape[2], w.shape[3]
    ws = (w.astype(_F32) * s.reshape(1, 1, 1, cout)).astype(_BF)
    idx, msk = _pack_idx(P, cin)
    g = ws.reshape(9 * cin, cout)[idx]
    g = jnp.where(msk[:, :, None], g, 0)
    return g.reshape(idx.shape[0], P * cout)


def _impl(x_nchw, w0c, s0, t0, w_c1, s1, t1, w_c2, s2, t2, w_c3, s3, t3,
          w0, b0, w1, b1, w2, b2):
    N = x_nchw.shape[0]
    x = jnp.transpose(x_nchw.astype(_BF), (0, 2, 3, 1))
    xp = jnp.pad(x, ((0, 0), (1, 1), (1, 1), (0, 0)))
    xr = xp.reshape(N, 98, 294)

    wp = {
        "w0": _pack_conv(w0c.reshape(3, 3, 3, 16), 16, s0),
        "w1": _pack_conv(w_c1, 8, s1),
        "w2": _pack_conv(w_c2, 4, s2),
        "w3": _pack_conv(w_c3, 2, s3),
        "t0": jnp.tile(t0, (1, 16)), "t1": jnp.tile(t1, (1, 8)),
        "t2": jnp.tile(t2, (1, 4)), "t3": jnp.tile(t3, (1, 2)),
    }
    feat = _conv_tower(xr, wp, N).reshape(N, 36 * 128)
    return _head(feat, w0, b0, w1, b1, w2, b2)


_forward = jax.jit(_impl)


def kernel(x_nchw, w0c, s0, t0, w_c1, s1, t1, w_c2, s2, t2, w_c3, s3, t3,
           w0, b0, w1, b1, w2, b2):
    return _forward(x_nchw, w0c, s0, t0, w_c1, s1, t1, w_c2, s2, t2,
                    w_c3, s3, t3, w0, b0, w1, b1, w2, b2)


# speedup vs baseline: 1.7087x; 1.2283x over previous
import jax
import jax.numpy as jnp
import numpy as np
from jax.experimental import pallas as pl
from jax.experimental.pallas import tpu as pltpu

_BF = jnp.bfloat16
_F32 = jnp.float32
_B = 16


def _tower_kernel(xr_ref,
                  w0_ref, t0_ref, w1_ref, t1_ref,
                  w2_ref, t2_ref, w3_ref, t3_ref,
                  out_ref, *scr):
    for c in range(2):
        lhs0, lhs1, lhs2, lhs3, ys, pad1, pad2, pad3 = scr[8 * c:8 * c + 8]
        pad1[...] = jnp.zeros_like(pad1)
        pad2[...] = jnp.zeros_like(pad2)
        pad3[...] = jnp.zeros_like(pad3)

        def conv_pool(lhs_s, w_ref, t_ref, H, K):
            M = 48 * H
            y = jnp.dot(lhs_s[...].reshape(M, K), w_ref[...],
                        preferred_element_type=_F32)
            y = jnp.maximum(y + t_ref[...], 0.0)
            ys[0:M, :] = jnp.maximum(y[:, :128], y[:, 128:])
            a = jnp.maximum(ys[pl.ds(0, M // 2, 2), :],
                            ys[pl.ds(1, M // 2, 2), :])
            return a.astype(_BF).reshape(8, 6, H // 2, 128)

        for dy in range(3):
            for g in range(6):
                lhs0[:, g, :, dy * 54:(dy + 1) * 54] = \
                    xr_ref[8 * c:8 * c + 8, dy:dy + 96, 48 * g:48 * g + 54]
        br = conv_pool(lhs0, w0_ref, t0_ref, 96, 162)
        pad1[:, 1:49, 16:784] = jnp.concatenate(
            [br[:, i] for i in range(6)], axis=-1)

        for dy in range(3):
            for g in range(6):
                lhs1[:, g, :, dy * 160:(dy + 1) * 160] = \
                    pad1[:, dy:dy + 48, 128 * g:128 * g + 160]
        br = conv_pool(lhs1, w1_ref, t1_ref, 48, 480)
        pad2[:, 1:25, 32:800] = jnp.concatenate(
            [br[:, i] for i in range(6)], axis=-1)

        for dy in range(3):
            lhs2[:, :, :, dy * 256 + 192:(dy + 1) * 256] = \
                jnp.zeros((8, 6, 24, 64), _BF)
            for g in range(6):
                lhs2[:, g, :, dy * 256:dy * 256 + 192] = \
                    pad2[:, dy:dy + 24, 128 * g:128 * g + 192]
        br = conv_pool(lhs2, w2_ref, t2_ref, 24, 768)
        pad3[:, 1:13, 64:832] = jnp.concatenate(
            [br[:, i] for i in range(6)], axis=-1)

        for dy in range(3):
            for g in range(6):
                lhs3[:, g, :, dy * 256:(dy + 1) * 256] = \
                    pad3[:, dy:dy + 12, 128 * g:128 * g + 256]
        br = conv_pool(lhs3, w3_ref, t3_ref, 12, 768)
        out_ref[8 * c:8 * c + 8] = jnp.transpose(
            br, (0, 2, 1, 3)).reshape(8, 36, 128)


def _pair_scratches():
    return [
        pltpu.VMEM((8, 6, 96, 162), _BF),
        pltpu.VMEM((8, 6, 48, 480), _BF),
        pltpu.VMEM((8, 6, 24, 768), _BF),
        pltpu.VMEM((8, 6, 12, 768), _BF),
        pltpu.VMEM((4608, 128), _F32),
        pltpu.VMEM((8, 50, 800), _BF),
        pltpu.VMEM((8, 26, 832), _BF),
        pltpu.VMEM((8, 14, 896), _BF),
    ]


def _conv_tower(xr, wp, N):
    const2 = lambda n: (0, 0)
    in_specs = [pl.BlockSpec((_B, 98, 294), lambda n: (n, 0, 0))]
    for i in range(4):
        in_specs += [pl.BlockSpec(wp[f"w{i}"].shape, const2),
                     pl.BlockSpec((1, 256), const2)]
    return pl.pallas_call(
        _tower_kernel,
        out_shape=jax.ShapeDtypeStruct((N, 36, 128), _BF),
        grid=(N // _B,),
        in_specs=in_specs,
        out_specs=pl.BlockSpec((_B, 36, 128), lambda n: (n, 0, 0)),
        scratch_shapes=_pair_scratches() + _pair_scratches(),
        compiler_params=pltpu.CompilerParams(
            dimension_semantics=("parallel",),
            vmem_limit_bytes=32 * 1024 * 1024),
    )(xr, wp["w0"], wp["t0"], wp["w1"], wp["t1"],
      wp["w2"], wp["t2"], wp["w3"], wp["t3"])


def _head_kernel(x_ref, w0_ref, b0_ref, w1_ref, b1_ref, w2_ref, b2_ref,
                 o_ref):
    h = jnp.maximum(
        jnp.dot(x_ref[...], w0_ref[...], preferred_element_type=_F32)
        + b0_ref[...], 0.0)
    h = jnp.maximum(
        jnp.dot(h.astype(_BF), w1_ref[...], preferred_element_type=_F32)
        + b1_ref[...], 0.0)
    z = jnp.dot(h.astype(_BF), w2_ref[...], preferred_element_type=_F32) \
        + b2_ref[...]
    o_ref[...] = 1.0 / (1.0 + jnp.exp(-z))


def _head(feat, w0, b0, w1, b1, w2, b2):
    N = feat.shape[0]
    vmem = pl.BlockSpec(memory_space=pltpu.MemorySpace.VMEM)
    return pl.pallas_call(
        _head_kernel,
        out_shape=jax.ShapeDtypeStruct((N, 1), _F32),
        in_specs=[vmem] * 7,
        out_specs=vmem,
    )(feat, w0, b0, w1, b1, w2, b2)


def _pack_idx(P, cin, seg):
    L = P + 2
    R = 3 * seg
    idx = np.zeros((R, P), np.int32)
    msk = np.zeros((R, P), np.bool_)
    order = list(range(0, P, 2)) + list(range(1, P, 2))
    for col, p in enumerate(order):
        for dy in range(3):
            for dx in range(3):
                for ci in range(cin):
                    r = dy * seg + (p + dx) * cin + ci
                    idx[r, col] = (dy * 3 + dx) * cin + ci
                    msk[r, col] = True
    return jnp.asarray(idx), jnp.asarray(msk)


def _pack_conv(w, P, s, seg=None):
    cin, cout = w.shape[2], w.shape[3]
    seg = seg or (P + 2) * cin
    ws = (w.astype(_F32) * s.reshape(1, 1, 1, cout)).astype(_BF)
    idx, msk = _pack_idx(P, cin, seg)
    g = ws.reshape(9 * cin, cout)[idx]
    g = jnp.where(msk[:, :, None], g, 0)
    return g.reshape(idx.shape[0], P * cout)


def _impl(x_nchw, w0c, s0, t0, w_c1, s1, t1, w_c2, s2, t2, w_c3, s3, t3,
          w0, b0, w1, b1, w2, b2):
    N = x_nchw.shape[0]
    x = jnp.transpose(x_nchw.astype(_BF), (0, 2, 3, 1))
    xp = jnp.pad(x, ((0, 0), (1, 1), (1, 1), (0, 0)))
    xr = xp.reshape(N, 98, 294)

    wp = {
        "w0": _pack_conv(w0c.reshape(3, 3, 3, 16), 16, s0),
        "w1": _pack_conv(w_c1, 8, s1),
        "w2": _pack_conv(w_c2, 4, s2, seg=256),
        "w3": _pack_conv(w_c3, 2, s3),
        "t0": jnp.tile(t0, (1, 16)), "t1": jnp.tile(t1, (1, 8)),
        "t2": jnp.tile(t2, (1, 4)), "t3": jnp.tile(t3, (1, 2)),
    }
    feat = _conv_tower(xr, wp, N).reshape(N, 36 * 128)
    return _head(feat, w0, b0, w1, b1, w2, b2)


_forward = jax.jit(_impl)


def kernel(x_nchw, w0c, s0, t0, w_c1, s1, t1, w_c2, s2, t2, w_c3, s3, t3,
           w0, b0, w1, b1, w2, b2):
    return _forward(x_nchw, w0c, s0, t0, w_c1, s1, t1, w_c2, s2, t2,
                    w_c3, s3, t3, w0, b0, w1, b1, w2, b2)
